# baseline (speedup 1.0000x reference)
import functools

import jax
import jax.numpy as jnp
from jax import lax
from jax.experimental import pallas as pl
from jax.experimental.pallas import tpu as pltpu
from jax.experimental.pallas import tpu_sc as plsc

N = 10000
NP = 10240
D = 128
E = 320000
CH = 128
CPT = 160
EP = 16 * CPT * CH
EPT = CPT * CH

_MESH = plsc.VectorSubcoreMesh(core_axis_name="c", subcore_axis_name="s")


def _stage_a_body(feat_hbm, perm_hbm, degidx_hbm, z1_hbm,
                  xp_hbm, degout_hbm,
                  ia, ib, onesbuf, pidx, rowbuf, deg_sp, sma, smb, sem):
    c = lax.axis_index("c")
    s = lax.axis_index("s")
    w = c * 16 + s
    pltpu.sync_copy(z1_hbm, deg_sp.at[pl.ds(s * 1280, 1280)])
    for k in range(8):
        onesbuf[pl.ds(16 * k, 16)] = jnp.ones((16,), jnp.float32)
    plsc.subcore_barrier()

    base = w * EPT
    pltpu.async_copy(degidx_hbm.at[pl.ds(base, CH)], ia, sma)
    pltpu.async_copy(degidx_hbm.at[pl.ds(base + CH, CH)], ib, smb)

    def deg_body(i, _):
        na = jnp.minimum(2 * i + 2, CPT - 1)
        nb = jnp.minimum(2 * i + 3, CPT - 1)
        pltpu.make_async_copy(degidx_hbm.at[pl.ds(base, CH)], ia, sma).wait()
        pltpu.sync_copy(onesbuf, deg_sp.at[ia], add=True)
        pltpu.async_copy(degidx_hbm.at[pl.ds(base + na * CH, CH)], ia, sma)
        pltpu.make_async_copy(degidx_hbm.at[pl.ds(base, CH)], ib, smb).wait()
        pltpu.sync_copy(onesbuf, deg_sp.at[ib], add=True)
        pltpu.async_copy(degidx_hbm.at[pl.ds(base + nb * CH, CH)], ib, smb)
        return 0
    lax.fori_loop(0, CPT // 2, deg_body, 0)
    pltpu.make_async_copy(degidx_hbm.at[pl.ds(base, CH)], ia, sma).wait()
    pltpu.make_async_copy(degidx_hbm.at[pl.ds(base, CH)], ib, smb).wait()

    def gat_body(k, _):
        rbase = w * 320 + k * 80
        pltpu.sync_copy(perm_hbm.at[pl.ds(rbase, 80)], pidx)
        pltpu.async_copy(feat_hbm.at[pidx], rowbuf, sem).wait()
        pltpu.sync_copy(rowbuf, xp_hbm.at[pl.ds(rbase, 80)])
        return 0
    lax.fori_loop(0, 4, gat_body, 0)

    plsc.subcore_barrier()
    pltpu.sync_copy(deg_sp.at[pl.ds(s * 1280, 1280)],
                    degout_hbm.at[pl.ds(c * 2 * NP + s * 1280, 1280)])


_stage_a = functools.partial(
    pl.kernel,
    out_type=[jax.ShapeDtypeStruct((NP, D), jnp.float32),
              jax.ShapeDtypeStruct((2 * 2 * NP,), jnp.float32)],
    mesh=_MESH,
    scratch_types=[
        pltpu.VMEM((CH,), jnp.int32),
        pltpu.VMEM((CH,), jnp.int32),
        pltpu.VMEM((CH,), jnp.float32),
        pltpu.VMEM((80,), jnp.int32),
        pltpu.VMEM((80, D), jnp.float32),
        pltpu.VMEM_SHARED((2 * NP,), jnp.float32),
        pltpu.SemaphoreType.DMA,
        pltpu.SemaphoreType.DMA,
        pltpu.SemaphoreType.DMA,
    ],
)(_stage_a_body)


def _stage_b_body(feat_ref, xp_ref, deg_ref, out_ref):
    h = pl.program_id(0)
    x = jnp.where(h == 0, feat_ref[...], xp_ref[...])
    d = deg_ref[0] + deg_ref[1]
    nsrc = lax.rsqrt(jnp.maximum(d, 1.0))
    out_ref[...] = x * nsrc


def _stage_b(feat_p, xp, degcol):
    return pl.pallas_call(
        _stage_b_body,
        grid=(2, 8),
        in_specs=[
            pl.BlockSpec((1280, D), lambda h, j: (j, 0)),
            pl.BlockSpec((1280, D), lambda h, j: (j, 0)),
            pl.BlockSpec((2, 1280, 1), lambda h, j: (0, j, 0)),
        ],
        out_specs=pl.BlockSpec((1280, D), lambda h, j: (h * 8 + j, 0)),
        out_shape=jax.ShapeDtypeStruct((2 * NP, D), jnp.float32),
    )(feat_p, xp, degcol)


def _stage_c_body(h2_hbm, src2_hbm, dst_hbm, z2_hbm,
                  out_hbm,
                  siA0, siA1, diA0, diA1, siB0, siB1, diB0, diB1,
                  r0, r1, acc_sp, g0, g1, ixa, ixb):
    c = lax.axis_index("c")
    s = lax.axis_index("s")

    def zbody(k, _):
        pltpu.sync_copy(z2_hbm, acc_sp.at[pl.ds(s * 640 + k * CH, CH)])
        return 0
    lax.fori_loop(0, 5, zbody, 0)
    plsc.subcore_barrier()

    sbase = c * EP + s * EPT
    dbase = s * EPT

    def ldsi(g, buf, sem):
        return pltpu.async_copy(src2_hbm.at[pl.ds(sbase + g * CH, CH)], buf, sem)

    def lddi(g, buf, sem):
        return pltpu.async_copy(dst_hbm.at[pl.ds(dbase + g * CH, CH)], buf, sem)

    def widx(buf, sem):
        pltpu.make_async_copy(src2_hbm.at[pl.ds(sbase, CH)], buf, sem).wait()

    pltpu.sync_copy(src2_hbm.at[pl.ds(sbase, CH)], siA0)
    pltpu.sync_copy(dst_hbm.at[pl.ds(dbase, CH)], diA0)
    pltpu.sync_copy(src2_hbm.at[pl.ds(sbase + CH, CH)], siA1)
    pltpu.sync_copy(dst_hbm.at[pl.ds(dbase + CH, CH)], diA1)
    pltpu.async_copy(h2_hbm.at[siA0], r0, g0)
    pltpu.async_copy(h2_hbm.at[siA1], r1, g1)
    ldsi(2, siB0, ixb)
    lddi(2, diB0, ixb)
    ldsi(3, siB1, ixb)
    lddi(3, diB1, ixb)

    def body(i, _):
        pltpu.make_async_copy(h2_hbm.at[siA0], r0, g0).wait()
        pltpu.sync_copy(r0, acc_sp.at[diA0], add=True)
        for _k in range(4):
            widx(siB0, ixb)
        pltpu.async_copy(h2_hbm.at[siB0], r0, g0)
        pltpu.make_async_copy(h2_hbm.at[siA1], r1, g1).wait()
        pltpu.sync_copy(r1, acc_sp.at[diA1], add=True)
        na = jnp.minimum(4 * i + 4, CPT - 1)
        nb = jnp.minimum(4 * i + 5, CPT - 1)
        ldsi(na, siA0, ixa)
        lddi(na, diA0, ixa)
        ldsi(nb, siA1, ixa)
        lddi(nb, diA1, ixa)
        pltpu.async_copy(h2_hbm.at[siB1], r1, g1)
        pltpu.make_async_copy(h2_hbm.at[siB0], r0, g0).wait()
        pltpu.sync_copy(r0, acc_sp.at[diB0], add=True)
        for _k in range(4):
            widx(siA0, ixa)
        pltpu.async_copy(h2_hbm.at[siA0], r0, g0)
        pltpu.make_async_copy(h2_hbm.at[siB1], r1, g1).wait()
        pltpu.sync_copy(r1, acc_sp.at[diB1], add=True)
        nc = jnp.minimum(4 * i + 6, CPT - 1)
        nd = jnp.minimum(4 * i + 7, CPT - 1)
        ldsi(nc, siB0, ixb)
        lddi(nc, diB0, ixb)
        ldsi(nd, siB1, ixb)
        lddi(nd, diB1, ixb)
        pltpu.async_copy(h2_hbm.at[siA1], r1, g1)
        return 0
    lax.fori_loop(0, CPT // 4, body, 0)

    pltpu.make_async_copy(h2_hbm.at[siA0], r0, g0).wait()
    pltpu.make_async_copy(h2_hbm.at[siA1], r1, g1).wait()
    for _k in range(4):
        widx(siB0, ixb)
    plsc.subcore_barrier()

    def rbody(k, _):
        rb = s * 640 + k * CH
        pltpu.sync_copy(acc_sp.at[pl.ds(rb, CH)], r0)
        pltpu.sync_copy(r0, out_hbm.at[pl.ds(c * NP + rb, CH)])
        return 0
    lax.fori_loop(0, 5, rbody, 0)


_stage_c = functools.partial(
    pl.kernel,
    out_type=jax.ShapeDtypeStruct((2 * NP, D), jnp.float32),
    mesh=_MESH,
    scratch_types=[
        pltpu.VMEM((CH,), jnp.int32),
        pltpu.VMEM((CH,), jnp.int32),
        pltpu.VMEM((CH,), jnp.int32),
        pltpu.VMEM((CH,), jnp.int32),
        pltpu.VMEM((CH,), jnp.int32),
        pltpu.VMEM((CH,), jnp.int32),
        pltpu.VMEM((CH,), jnp.int32),
        pltpu.VMEM((CH,), jnp.int32),
        pltpu.VMEM((CH, D), jnp.float32),
        pltpu.VMEM((CH, D), jnp.float32),
        pltpu.VMEM_SHARED((NP, D), jnp.float32),
        pltpu.SemaphoreType.DMA, pltpu.SemaphoreType.DMA,
        pltpu.SemaphoreType.DMA, pltpu.SemaphoreType.DMA,
    ],
)(_stage_c_body)


def _stage_d_body(agg_ref, deg_ref, w_ref, b_ref, wd_ref, out_ref):
    dp = deg_ref[...]
    d = dp[0] + dp[1]
    ndst = lax.rsqrt(jnp.maximum(d[NP:NP + N], 1.0))
    W = w_ref[...]
    b = b_ref[...]
    Wd = wd_ref[...]
    pos_a = agg_ref[0:N, :] * ndst
    neg_a = agg_ref[NP:NP + N, :] * ndst
    pos = jnp.maximum(jnp.dot(pos_a, W, preferred_element_type=jnp.float32) + b, 0.0)
    neg = jnp.maximum(jnp.dot(neg_a, W, preferred_element_type=jnp.float32) + b, 0.0)
    summary = jax.nn.sigmoid(jnp.mean(pos, axis=0, keepdims=True))
    p2 = jnp.dot(pos, Wd, preferred_element_type=jnp.float32)
    n2 = jnp.dot(neg, Wd, preferred_element_type=jnp.float32)
    pos_logit = jnp.sum(p2 * summary, axis=1, keepdims=True)
    neg_logit = jnp.sum(n2 * summary, axis=1, keepdims=True)

    def softplus(x):
        return jnp.maximum(x, 0.0) + jnp.log(1.0 + jnp.exp(-jnp.abs(x)))

    l1 = jnp.mean(softplus(-pos_logit))
    l2 = jnp.mean(softplus(neg_logit))
    out_ref[...] = jnp.broadcast_to(l1 + l2, (1, 1))


def _stage_d(agg, degcol, W_gcn, b_row, W_disc):
    return pl.pallas_call(
        _stage_d_body,
        out_shape=jax.ShapeDtypeStruct((1, 1), jnp.float32),
    )(agg, degcol, W_gcn, b_row, W_disc)


def kernel(features, edge_index, perm, W_gcn, b_gcn, W_disc):
    src = edge_index[0]
    dst = edge_index[1]
    src_p = jnp.pad(src, (0, EP - E), constant_values=N)
    dst_p = jnp.pad(dst, (0, EP - E), constant_values=NP - 1)
    src2 = jnp.concatenate([src_p, src_p + NP])
    deg_idx = jnp.concatenate([src_p, dst_p + NP])
    perm_p = jnp.pad(perm, (0, NP - N))
    feat_p = jnp.pad(features, ((0, NP - N), (0, 0)))
    z1 = jnp.zeros((1280,), jnp.float32)
    z2 = jnp.zeros((CH, D), jnp.float32)

    xp, degout = _stage_a(feat_p, perm_p, deg_idx, z1)
    degcol = degout.reshape(2, 2 * NP, 1)
    h2 = _stage_b(feat_p, xp, degcol)
    agg = _stage_c(h2, src2, dst_p, z2)
    loss = _stage_d(agg, degcol, W_gcn, b_gcn.reshape(1, D), W_disc)
    return loss[0, 0]


# speedup vs baseline: 1.3400x; 1.3400x over previous
import functools

import jax
import jax.numpy as jnp
from jax import lax
from jax.experimental import pallas as pl
from jax.experimental.pallas import tpu as pltpu
from jax.experimental.pallas import tpu_sc as plsc

N = 10000
NP = 10240
D = 128
E = 320000
CH = 128
CPT = 158
EP = 16 * CPT * CH
EPT = CPT * CH

_MESH = plsc.VectorSubcoreMesh(core_axis_name="c", subcore_axis_name="s")


def _stage_a_body(feat_hbm, perm_hbm, degidx_hbm, z1_hbm,
                  xp_hbm, degout_hbm,
                  ia, ib, onesbuf, pidx, rowbuf, deg_sp, sma, smb, sem):
    c = lax.axis_index("c")
    s = lax.axis_index("s")
    w = c * 16 + s
    pltpu.sync_copy(z1_hbm, deg_sp.at[pl.ds(s * 1280, 1280)])
    for k in range(8):
        onesbuf[pl.ds(16 * k, 16)] = jnp.ones((16,), jnp.float32)
    plsc.subcore_barrier()

    base = w * EPT
    pltpu.async_copy(degidx_hbm.at[pl.ds(base, CH)], ia, sma)
    pltpu.async_copy(degidx_hbm.at[pl.ds(base + CH, CH)], ib, smb)

    def deg_body(i, _):
        na = jnp.minimum(2 * i + 2, CPT - 1)
        nb = jnp.minimum(2 * i + 3, CPT - 1)
        pltpu.make_async_copy(degidx_hbm.at[pl.ds(base, CH)], ia, sma).wait()
        pltpu.sync_copy(onesbuf, deg_sp.at[ia], add=True)
        pltpu.async_copy(degidx_hbm.at[pl.ds(base + na * CH, CH)], ia, sma)
        pltpu.make_async_copy(degidx_hbm.at[pl.ds(base, CH)], ib, smb).wait()
        pltpu.sync_copy(onesbuf, deg_sp.at[ib], add=True)
        pltpu.async_copy(degidx_hbm.at[pl.ds(base + nb * CH, CH)], ib, smb)
        return 0
    lax.fori_loop(0, CPT // 2, deg_body, 0)
    pltpu.make_async_copy(degidx_hbm.at[pl.ds(base, CH)], ia, sma).wait()
    pltpu.make_async_copy(degidx_hbm.at[pl.ds(base, CH)], ib, smb).wait()

    def gat_body(k, _):
        rbase = w * 320 + k * 80
        pltpu.sync_copy(perm_hbm.at[pl.ds(rbase, 80)], pidx)
        pltpu.async_copy(feat_hbm.at[pidx], rowbuf, sem).wait()
        pltpu.sync_copy(rowbuf, xp_hbm.at[pl.ds(rbase, 80)])
        return 0
    lax.fori_loop(0, 4, gat_body, 0)

    plsc.subcore_barrier()
    pltpu.sync_copy(deg_sp.at[pl.ds(s * 1280, 1280)],
                    degout_hbm.at[pl.ds(c * 2 * NP + s * 1280, 1280)])


_stage_a = functools.partial(
    pl.kernel,
    out_type=[jax.ShapeDtypeStruct((NP, D), jnp.float32),
              jax.ShapeDtypeStruct((2 * 2 * NP,), jnp.float32)],
    mesh=_MESH,
    scratch_types=[
        pltpu.VMEM((CH,), jnp.int32),
        pltpu.VMEM((CH,), jnp.int32),
        pltpu.VMEM((CH,), jnp.float32),
        pltpu.VMEM((80,), jnp.int32),
        pltpu.VMEM((80, D), jnp.float32),
        pltpu.VMEM_SHARED((2 * NP,), jnp.float32),
        pltpu.SemaphoreType.DMA,
        pltpu.SemaphoreType.DMA,
        pltpu.SemaphoreType.DMA,
    ],
)(_stage_a_body)


def _stage_b_body(feat_ref, xp_ref, deg_ref, out_ref):
    h = pl.program_id(0)
    x = jnp.where(h == 0, feat_ref[...], xp_ref[...])
    d = deg_ref[0] + deg_ref[1]
    nsrc = lax.rsqrt(jnp.maximum(d, 1.0))
    out_ref[...] = x * nsrc


def _stage_b(feat_p, xp, degcol):
    return pl.pallas_call(
        _stage_b_body,
        grid=(2, 8),
        in_specs=[
            pl.BlockSpec((1280, D), lambda h, j: (j, 0)),
            pl.BlockSpec((1280, D), lambda h, j: (j, 0)),
            pl.BlockSpec((2, 1280, 1), lambda h, j: (0, j, 0)),
        ],
        out_specs=pl.BlockSpec((1280, D), lambda h, j: (h * 8 + j, 0)),
        out_shape=jax.ShapeDtypeStruct((2 * NP, D), jnp.float32),
    )(feat_p, xp, degcol)


def _stage_c_body(h2_hbm, src2_hbm, dst_hbm, z2_hbm,
                  out_hbm,
                  siA0, siA1, diA0, diA1,
                  r0, r1, acc_sp, g0, g1):
    c = lax.axis_index("c")
    s = lax.axis_index("s")

    def zbody(k, _):
        pltpu.sync_copy(z2_hbm, acc_sp.at[pl.ds(s * 640 + k * CH, CH)])
        return 0
    lax.fori_loop(0, 5, zbody, 0)
    plsc.subcore_barrier()

    sbase = c * EP + s * EPT
    dbase = s * EPT

    pltpu.sync_copy(src2_hbm.at[pl.ds(sbase, CH)], siA0)
    pltpu.sync_copy(dst_hbm.at[pl.ds(dbase, CH)], diA0)
    pltpu.async_copy(h2_hbm.at[siA0], r0, g0)
    pltpu.sync_copy(src2_hbm.at[pl.ds(sbase + CH, CH)], siA1)
    pltpu.sync_copy(dst_hbm.at[pl.ds(dbase + CH, CH)], diA1)
    pltpu.async_copy(h2_hbm.at[siA1], r1, g1)

    def body(i, _):
        pltpu.make_async_copy(h2_hbm.at[siA0], r0, g0).wait()
        pltpu.sync_copy(r0, acc_sp.at[diA0], add=True)
        nb = jnp.minimum(2 * i + 2, CPT - 1)
        pltpu.sync_copy(src2_hbm.at[pl.ds(sbase + nb * CH, CH)], siA0)
        pltpu.sync_copy(dst_hbm.at[pl.ds(dbase + nb * CH, CH)], diA0)
        pltpu.async_copy(h2_hbm.at[siA0], r0, g0)
        pltpu.make_async_copy(h2_hbm.at[siA1], r1, g1).wait()
        pltpu.sync_copy(r1, acc_sp.at[diA1], add=True)
        nb2 = jnp.minimum(2 * i + 3, CPT - 1)
        pltpu.sync_copy(src2_hbm.at[pl.ds(sbase + nb2 * CH, CH)], siA1)
        pltpu.sync_copy(dst_hbm.at[pl.ds(dbase + nb2 * CH, CH)], diA1)
        pltpu.async_copy(h2_hbm.at[siA1], r1, g1)
        return 0
    lax.fori_loop(0, CPT // 2, body, 0)

    pltpu.make_async_copy(h2_hbm.at[siA0], r0, g0).wait()
    pltpu.make_async_copy(h2_hbm.at[siA1], r1, g1).wait()
    plsc.subcore_barrier()

    def rbody(k, _):
        rb = s * 640 + k * CH
        pltpu.sync_copy(acc_sp.at[pl.ds(rb, CH)], r0)
        pltpu.sync_copy(r0, out_hbm.at[pl.ds(c * NP + rb, CH)])
        return 0
    lax.fori_loop(0, 5, rbody, 0)


_stage_c = functools.partial(
    pl.kernel,
    out_type=jax.ShapeDtypeStruct((2 * NP, D), jnp.float32),
    mesh=_MESH,
    scratch_types=[
        pltpu.VMEM((CH,), jnp.int32),
        pltpu.VMEM((CH,), jnp.int32),
        pltpu.VMEM((CH,), jnp.int32),
        pltpu.VMEM((CH,), jnp.int32),
        pltpu.VMEM((CH, D), jnp.float32),
        pltpu.VMEM((CH, D), jnp.float32),
        pltpu.VMEM_SHARED((NP, D), jnp.float32),
        pltpu.SemaphoreType.DMA, pltpu.SemaphoreType.DMA,
    ],
)(_stage_c_body)


def _stage_d_body(agg_ref, deg_ref, w_ref, b_ref, wd_ref, out_ref):
    dp = deg_ref[...]
    d = dp[0] + dp[1]
    ndst = lax.rsqrt(jnp.maximum(d[NP:NP + N], 1.0))
    W = w_ref[...]
    b = b_ref[...]
    Wd = wd_ref[...]
    pos_a = agg_ref[0:N, :] * ndst
    neg_a = agg_ref[NP:NP + N, :] * ndst
    pos = jnp.maximum(jnp.dot(pos_a, W, preferred_element_type=jnp.float32) + b, 0.0)
    neg = jnp.maximum(jnp.dot(neg_a, W, preferred_element_type=jnp.float32) + b, 0.0)
    summary = jax.nn.sigmoid(jnp.mean(pos, axis=0, keepdims=True))
    p2 = jnp.dot(pos, Wd, preferred_element_type=jnp.float32)
    n2 = jnp.dot(neg, Wd, preferred_element_type=jnp.float32)
    pos_logit = jnp.sum(p2 * summary, axis=1, keepdims=True)
    neg_logit = jnp.sum(n2 * summary, axis=1, keepdims=True)

    def softplus(x):
        return jnp.maximum(x, 0.0) + jnp.log(1.0 + jnp.exp(-jnp.abs(x)))

    l1 = jnp.mean(softplus(-pos_logit))
    l2 = jnp.mean(softplus(neg_logit))
    out_ref[...] = jnp.broadcast_to(l1 + l2, (1, 1))


def _stage_d(agg, degcol, W_gcn, b_row, W_disc):
    return pl.pallas_call(
        _stage_d_body,
        out_shape=jax.ShapeDtypeStruct((1, 1), jnp.float32),
    )(agg, degcol, W_gcn, b_row, W_disc)


def kernel(features, edge_index, perm, W_gcn, b_gcn, W_disc):
    src = edge_index[0]
    dst = edge_index[1]
    src_p = jnp.pad(src, (0, EP - E), constant_values=N)
    dst_p = jnp.pad(dst, (0, EP - E), constant_values=NP - 1)
    src2 = jnp.concatenate([src_p, src_p + NP])
    deg_idx = jnp.concatenate([src_p, dst_p + NP])
    perm_p = jnp.pad(perm, (0, NP - N))
    feat_p = jnp.pad(features, ((0, NP - N), (0, 0)))
    z1 = jnp.zeros((1280,), jnp.float32)
    z2 = jnp.zeros((CH, D), jnp.float32)

    xp, degout = _stage_a(feat_p, perm_p, deg_idx, z1)
    degcol = degout.reshape(2, 2 * NP, 1)
    h2 = _stage_b(feat_p, xp, degcol)
    agg = _stage_c(h2, src2, dst_p, z2)
    loss = _stage_d(agg, degcol, W_gcn, b_gcn.reshape(1, D), W_disc)
    return loss[0, 0]
